# baseline (speedup 1.0000x reference)
import jax
import jax.numpy as jnp
from jax.experimental import pallas as pl

_NEG_INF = float("-inf")


def _gate_body(x_ref, wt_ref, b_ref, idx_ref, score_ref):
    x = x_ref[...]
    wt = wt_ref[...]
    g = jnp.dot(x, wt, preferred_element_type=jnp.float32) + b_ref[...]
    e = g.shape[1]
    eidx = jax.lax.broadcasted_iota(jnp.int32, g.shape, 1)
    m1 = jnp.max(g, axis=1, keepdims=True)
    i1 = jnp.min(jnp.where(g == m1, eidx, e), axis=1, keepdims=True)
    g2 = jnp.where(eidx == i1, _NEG_INF, g)
    m2 = jnp.max(g2, axis=1, keepdims=True)
    i2 = jnp.min(jnp.where(g2 == m2, eidx, e), axis=1, keepdims=True)
    idx_ref[...] = jnp.concatenate([i1, i2], axis=1)
    e2 = jnp.exp(m2 - m1)
    d = 1.0 / (1.0 + e2)
    score_ref[...] = jnp.concatenate([d, e2 * d], axis=1)


def kernel(inp, W, b):
    m, dm = inp.shape
    e = W.shape[0]
    bm = 512
    grid = (m // bm,)
    wt = W.T
    b2 = b.reshape(1, e)
    idx, score = pl.pallas_call(
        _gate_body,
        grid=grid,
        in_specs=[
            pl.BlockSpec((bm, dm), lambda i: (i, 0)),
            pl.BlockSpec((dm, e), lambda i: (0, 0)),
            pl.BlockSpec((1, e), lambda i: (0, 0)),
        ],
        out_specs=[
            pl.BlockSpec((bm, 2), lambda i: (i, 0)),
            pl.BlockSpec((bm, 2), lambda i: (i, 0)),
        ],
        out_shape=[
            jax.ShapeDtypeStruct((m, 2), jnp.int32),
            jax.ShapeDtypeStruct((m, 2), jnp.float32),
        ],
    )(inp, wt, b2)
    return idx, score


# speedup vs baseline: 2.8516x; 2.8516x over previous
import jax
import jax.numpy as jnp
from jax.experimental import pallas as pl

_NEG_INF = float("-inf")


def _gate_body(x_ref, wt_ref, b_ref, idx_ref, score_ref):
    x = x_ref[...]
    wt = wt_ref[...]
    g = jnp.dot(x, wt, preferred_element_type=jnp.float32)
    gt = g.T + b_ref[...]
    e = gt.shape[0]
    eidx = jax.lax.broadcasted_iota(jnp.int32, gt.shape, 0)
    m1 = jnp.max(gt, axis=0, keepdims=True)
    i1 = jnp.min(jnp.where(gt == m1, eidx, e), axis=0, keepdims=True)
    g2 = jnp.where(eidx == i1, _NEG_INF, gt)
    m2 = jnp.max(g2, axis=0, keepdims=True)
    i2 = jnp.min(jnp.where(g2 == m2, eidx, e), axis=0, keepdims=True)
    idx_ref[...] = jnp.concatenate([i1, i2], axis=0)
    e2 = jnp.exp(m2 - m1)
    d = 1.0 / (1.0 + e2)
    score_ref[...] = jnp.concatenate([d, e2 * d], axis=0)


def kernel(inp, W, b):
    m, dm = inp.shape
    e = W.shape[0]
    bm = 2048
    grid = (m // bm,)
    wt = W.T
    b2 = b.reshape(e, 1)
    idx_t, score_t = pl.pallas_call(
        _gate_body,
        grid=grid,
        in_specs=[
            pl.BlockSpec((bm, dm), lambda i: (i, 0)),
            pl.BlockSpec((dm, e), lambda i: (0, 0)),
            pl.BlockSpec((e, 1), lambda i: (0, 0)),
        ],
        out_specs=[
            pl.BlockSpec((2, bm), lambda i: (0, i)),
            pl.BlockSpec((2, bm), lambda i: (0, i)),
        ],
        out_shape=[
            jax.ShapeDtypeStruct((2, m), jnp.int32),
            jax.ShapeDtypeStruct((2, m), jnp.float32),
        ],
    )(inp, wt, b2)
    return idx_t.T, score_t.T


# speedup vs baseline: 2.8600x; 1.0029x over previous
import jax
import jax.numpy as jnp
from jax.experimental import pallas as pl

_NEG_INF = float("-inf")


def _gate_body(x_ref, wt_ref, b_ref, idx_ref, score_ref):
    x = x_ref[...]
    wt = wt_ref[...]
    g = jnp.dot(x, wt, preferred_element_type=jnp.float32)
    gt = g.T + b_ref[...]
    e = gt.shape[0]
    eidx = jax.lax.broadcasted_iota(jnp.int32, gt.shape, 0)
    m1 = jnp.max(gt, axis=0, keepdims=True)
    i1 = jnp.min(jnp.where(gt == m1, eidx, e), axis=0, keepdims=True)
    g2 = jnp.where(eidx == i1, _NEG_INF, gt)
    m2 = jnp.max(g2, axis=0, keepdims=True)
    i2 = jnp.min(jnp.where(g2 == m2, eidx, e), axis=0, keepdims=True)
    idx_ref[...] = jnp.concatenate([i1, i2], axis=0)
    e2 = jnp.exp(m2 - m1)
    d = 1.0 / (1.0 + e2)
    score_ref[...] = jnp.concatenate([d, e2 * d], axis=0)


def kernel(inp, W, b):
    m, dm = inp.shape
    e = W.shape[0]
    bm = 4096
    grid = (m // bm,)
    wt = W.T
    b2 = b.reshape(e, 1)
    idx_t, score_t = pl.pallas_call(
        _gate_body,
        grid=grid,
        in_specs=[
            pl.BlockSpec((bm, dm), lambda i: (i, 0)),
            pl.BlockSpec((dm, e), lambda i: (0, 0)),
            pl.BlockSpec((e, 1), lambda i: (0, 0)),
        ],
        out_specs=[
            pl.BlockSpec((2, bm), lambda i: (0, i)),
            pl.BlockSpec((2, bm), lambda i: (0, i)),
        ],
        out_shape=[
            jax.ShapeDtypeStruct((2, m), jnp.int32),
            jax.ShapeDtypeStruct((2, m), jnp.float32),
        ],
    )(inp, wt, b2)
    return idx_t.T, score_t.T
